# baseline (speedup 1.0000x reference)
import functools

import jax
import jax.numpy as jnp
from jax import lax
from jax.experimental import pallas as pl
from jax.experimental.pallas import tpu as pltpu
from jax.experimental.pallas import tpu_sc as plsc

ROWS = 32768
DIM = 768
NE = 64
BLK = 4096

NW = 32
RPW = ROWS // NW
GRP = RPW // 16


def _tc_body(x_ref, w_ref, b_ref, probs_ref, probst_ref):
    x = x_ref[...]
    w = w_ref[...]
    st = jax.lax.dot_general(w, x, (((1,), (1,)), ((), ())),
                             preferred_element_type=jnp.float32)
    st = st + b_ref[...]
    m = jnp.max(st, axis=0, keepdims=True)
    e = jnp.exp(st - m)
    probst = e / jnp.sum(e, axis=0, keepdims=True)
    probst_ref[...] = probst
    probs_ref[...] = probst.T


def _tc_probs(x, w, bc):
    return pl.pallas_call(
        _tc_body,
        grid=(ROWS // BLK,),
        in_specs=[
            pl.BlockSpec((BLK, DIM), lambda i: (i, 0)),
            pl.BlockSpec((NE, DIM), lambda i: (0, 0)),
            pl.BlockSpec((NE, 1), lambda i: (0, 0)),
        ],
        out_specs=[
            pl.BlockSpec((BLK, NE), lambda i: (i, 0)),
            pl.BlockSpec((NE, BLK), lambda i: (0, i)),
        ],
        out_shape=[
            jax.ShapeDtypeStruct((ROWS, NE), jnp.float32),
            jax.ShapeDtypeStruct((NE, ROWS), jnp.float32),
        ],
    )(x, w, bc)


CH = RPW // 4


@functools.partial(
    pl.kernel,
    out_type=jax.ShapeDtypeStruct((2, ROWS), jnp.int32),
    mesh=plsc.VectorSubcoreMesh(core_axis_name="c", subcore_axis_name="s"),
    scratch_types=[
        pltpu.VMEM((2, NE, CH), jnp.float32),
        pltpu.VMEM((2, RPW), jnp.int32),
        pltpu.SemaphoreType.DMA,
        pltpu.SemaphoreType.DMA,
    ],
)
def _sc_top2(probst_hbm, idx_hbm, pt_v, idx_v, sem0, sem1):
    wid = lax.axis_index("s") * 2 + lax.axis_index("c")
    base = wid * RPW
    sems = [sem0, sem1]

    def copy_chunk(c):
        return pltpu.make_async_copy(
            probst_hbm.at[:, pl.ds(base + c * CH, CH)],
            pt_v.at[c % 2], sems[c % 2])

    def one_group(bi, coff, off):
        m1 = jnp.full((16,), -1.0, jnp.float32)
        m2 = jnp.full((16,), -1.0, jnp.float32)
        i1 = jnp.zeros((16,), jnp.int32)
        i2 = jnp.zeros((16,), jnp.int32)
        for e in range(NE):
            v = pt_v[bi, e, pl.ds(off, 16)]
            col = jnp.full((16,), e, jnp.int32)
            gt1 = v > m1
            gt2 = v > m2
            m2 = jnp.where(gt1, m1, jnp.where(gt2, v, m2))
            i2 = jnp.where(gt1, i1, jnp.where(gt2, col, i2))
            m1 = jnp.where(gt1, v, m1)
            i1 = jnp.where(gt1, col, i1)
        idx_v[0, pl.ds(coff + off, 16)] = i1
        idx_v[1, pl.ds(coff + off, 16)] = i2

    copy_chunk(0).start()
    for c in range(4):
        if c + 1 < 4:
            copy_chunk(c + 1).start()
        copy_chunk(c).wait()

        def group_body(g, carry, _c=c):
            one_group(_c % 2, _c * CH, g * 32)
            one_group(_c % 2, _c * CH, g * 32 + 16)
            return carry

        lax.fori_loop(0, CH // 32, group_body, 0)

    pltpu.sync_copy(idx_v, idx_hbm.at[:, pl.ds(base, RPW)])


def _idx_t_body(idxt_ref, idx_ref):
    idx_ref[...] = idxt_ref[...].T


def _idx_transpose(idxt):
    return pl.pallas_call(
        _idx_t_body,
        grid=(4,),
        in_specs=[pl.BlockSpec((2, ROWS // 4), lambda i: (0, i))],
        out_specs=pl.BlockSpec((ROWS // 4, 2), lambda i: (i, 0)),
        out_shape=jax.ShapeDtypeStruct((ROWS, 2), jnp.int32),
    )(idxt)


@jax.jit
def kernel(x, weight, bias):
    bc = bias.reshape(NE, 1)
    probs, probst = _tc_probs(x, weight, bc)
    idxt = _sc_top2(probst)
    return probs, _idx_transpose(idxt)


# speedup vs baseline: 1.2518x; 1.2518x over previous
import functools

import jax
import jax.numpy as jnp
from jax import lax
from jax.experimental import pallas as pl
from jax.experimental.pallas import tpu as pltpu
from jax.experimental.pallas import tpu_sc as plsc

ROWS = 32768
DIM = 768
NE = 64
BLK = 4096

NW = 32
RPW = ROWS // NW
GRP = RPW // 16


def _tc_body(x_ref, w_ref, b_ref, probs_ref, probst_ref):
    x = x_ref[...]
    w = w_ref[...]
    st = jax.lax.dot_general(w, x, (((1,), (1,)), ((), ())),
                             preferred_element_type=jnp.float32)
    st = st + b_ref[...]
    m = jnp.max(st, axis=0, keepdims=True)
    e = jnp.exp(st - m)
    probst = e / jnp.sum(e, axis=0, keepdims=True)
    probst_ref[...] = probst
    probs_ref[...] = probst.T


def _tc_probs(x, w, bc):
    return pl.pallas_call(
        _tc_body,
        grid=(ROWS // BLK,),
        in_specs=[
            pl.BlockSpec((BLK, DIM), lambda i: (i, 0)),
            pl.BlockSpec((NE, DIM), lambda i: (0, 0)),
            pl.BlockSpec((NE, 1), lambda i: (0, 0)),
        ],
        out_specs=[
            pl.BlockSpec((BLK, NE), lambda i: (i, 0)),
            pl.BlockSpec((NE, BLK), lambda i: (0, i)),
        ],
        out_shape=[
            jax.ShapeDtypeStruct((ROWS, NE), jnp.float32),
            jax.ShapeDtypeStruct((NE, ROWS), jnp.float32),
        ],
    )(x, w, bc)


CH = RPW // 4


@functools.partial(
    pl.kernel,
    out_type=jax.ShapeDtypeStruct((2, ROWS), jnp.int32),
    mesh=plsc.VectorSubcoreMesh(core_axis_name="c", subcore_axis_name="s"),
    scratch_types=[
        pltpu.VMEM((2, NE, CH), jnp.float32),
        pltpu.VMEM((2, RPW), jnp.int32),
        pltpu.SemaphoreType.DMA,
        pltpu.SemaphoreType.DMA,
    ],
)
def _sc_top2(probst_hbm, idx_hbm, pt_v, idx_v, sem0, sem1):
    wid = lax.axis_index("s") * 2 + lax.axis_index("c")
    base = wid * RPW
    sems = [sem0, sem1]

    def copy_chunk(c):
        return pltpu.make_async_copy(
            probst_hbm.at[:, pl.ds(base + c * CH, CH)],
            pt_v.at[c % 2], sems[c % 2])

    def one_group(bi, coff, off):
        m1 = jnp.full((16,), -1.0, jnp.float32)
        m2 = jnp.full((16,), -1.0, jnp.float32)
        i1 = jnp.zeros((16,), jnp.int32)
        i2 = jnp.zeros((16,), jnp.int32)
        for e in range(NE):
            v = pt_v[bi, e, pl.ds(off, 16)]
            col = jnp.full((16,), e, jnp.int32)
            gt1 = v > m1
            gt2 = v > m2
            m2 = jnp.where(gt1, m1, jnp.where(gt2, v, m2))
            i2 = jnp.where(gt1, i1, jnp.where(gt2, col, i2))
            m1 = jnp.where(gt1, v, m1)
            i1 = jnp.where(gt1, col, i1)
        idx_v[0, pl.ds(coff + off, 16)] = i1
        idx_v[1, pl.ds(coff + off, 16)] = i2

    copy_chunk(0).start()
    for c in range(4):
        if c + 1 < 4:
            copy_chunk(c + 1).start()
        copy_chunk(c).wait()

        def group_body(g, carry, _c=c):
            one_group(_c % 2, _c * CH, g * 32)
            one_group(_c % 2, _c * CH, g * 32 + 16)
            return carry

        lax.fori_loop(0, CH // 32, group_body, 0)

    pltpu.sync_copy(idx_v, idx_hbm.at[:, pl.ds(base, RPW)])


@jax.jit
def kernel(x, weight, bias):
    bc = bias.reshape(NE, 1)
    probs, probst = _tc_probs(x, weight, bc)
    idxt = _sc_top2(probst)
    return probs, idxt.T
